# baseline (speedup 1.0000x reference)
import functools

import jax
import jax.numpy as jnp
from jax import lax
from jax.experimental import pallas as pl
from jax.experimental.pallas import tpu as pltpu
from jax.experimental.pallas import tpu_sc as plsc

N = 1024
L = 256
B = 128
V = 100000
WD = 50
PD = 5
H = 230
R = 53
SPB = N // B
TOK = SPB * L

NL = N * L
HALVES = 8
NLH = NL // HALVES
BH = B // HALVES
NW = 32
TPW = NLH // NW
CH = 128
SR = 4
BLK = CH * SR
NB = TPW // BLK
NIR = TPW // CH

WROWS = NLH // 2
PROWS = NLH // 8


def _fire(block, wtab, ptab, idxw, idxp, bw, bp, sem):
    for r in range(SR):
        pltpu.async_copy(wtab.at[idxw.at[block * SR + r]],
                         bw.at[pl.ds(r * CH, CH)], sem)
        pltpu.async_copy(ptab.at[idxp.at[block * SR + r]],
                         bp.at[pl.ds(r * CH, CH)], sem)


def _drain(wtab, ptab, bw, bp, sem):
    for r in range(SR):
        pltpu.make_async_copy(wtab.at[pl.ds(0, CH)],
                              bw.at[pl.ds(r * CH, CH)], sem).wait()
        pltpu.make_async_copy(ptab.at[pl.ds(0, CH)],
                              bp.at[pl.ds(r * CH, CH)], sem).wait()


def _wb(wid, block, bw, bp, wg, pg):
    off = wid * TPW + block * BLK
    btc = off // TOK
    p = off % TOK
    rw = btc * (TOK // 2) + p % (TOK // 2)
    qw = p // (TOK // 2)
    pltpu.sync_copy(bw, wg.at[pl.ds(rw, BLK), pl.ds(qw * 64, 64)])
    rp = btc * (TOK // 8)
    q8 = p // (TOK // 8)
    pltpu.sync_copy(bp.at[pl.ds(0, 256)],
                    pg.at[pl.ds(rp, 256), pl.ds(q8 * 16, 16)])
    pltpu.sync_copy(bp.at[pl.ds(256, 256)],
                    pg.at[pl.ds(rp, 256), pl.ds((q8 + 1) * 16, 16)])


@functools.partial(
    pl.kernel,
    mesh=plsc.VectorSubcoreMesh(core_axis_name="c", subcore_axis_name="s"),
    compiler_params=pltpu.CompilerParams(use_tc_tiling_on_sc=False),
    out_type=[
        jax.ShapeDtypeStruct((WROWS, 128), jnp.float32),
        jax.ShapeDtypeStruct((PROWS, 128), jnp.float32),
    ],
    scratch_types=[
        pltpu.VMEM((NIR, CH), jnp.int32),
        pltpu.VMEM((NIR, CH), jnp.int32),
        pltpu.VMEM((BLK, 64), jnp.float32),
        pltpu.VMEM((BLK, 64), jnp.float32),
        pltpu.VMEM((BLK, 16), jnp.float32),
        pltpu.VMEM((BLK, 16), jnp.float32),
        pltpu.SemaphoreType.DMA,
    ],
)
def _sc_gather(wtab, ptab, xw, xp, wg, pg,
               idxw, idxp, bwa, bwb, bpa, bpb, sem):
    wid = lax.axis_index("s") * 2 + lax.axis_index("c")
    pltpu.sync_copy(xw.at[pl.ds(wid * NIR, NIR)], idxw)
    pltpu.sync_copy(xp.at[pl.ds(wid * NIR, NIR)], idxp)
    _fire(0, wtab, ptab, idxw, idxp, bwa, bpa, sem)

    def body(k, carry):
        _fire(2 * k + 1, wtab, ptab, idxw, idxp, bwb, bpb, sem)
        _drain(wtab, ptab, bwa, bpa, sem)
        _wb(wid, 2 * k, bwa, bpa, wg, pg)

        @pl.when(k < NB // 2 - 1)
        def _():
            _fire(2 * k + 2, wtab, ptab, idxw, idxp, bwa, bpa, sem)

        _drain(wtab, ptab, bwb, bpb, sem)
        _wb(wid, 2 * k + 1, bwb, bpb, wg, pg)
        return carry

    lax.fori_loop(0, NB // 2, body, 0)


def _tc_body(xrel_ref, wg_hbm, pg_hbm, mask_ref, w_ref, cb_ref,
             relw_ref, relwt_ref, relb_ref, out_ref, wgv, pgv, dsem):
    b = pl.program_id(0)
    slot = lax.rem(b, 2)
    nslot = lax.rem(b + 1, 2)

    def fetch(step, sl):
        pltpu.make_async_copy(wg_hbm.at[pl.ds(step * (TOK // 2), TOK // 2)],
                              wgv.at[sl], dsem.at[sl]).start()
        pltpu.make_async_copy(pg_hbm.at[pl.ds(step * (TOK // 8), TOK // 8)],
                              pgv.at[sl], dsem.at[sl]).start()

    def drain(step, sl):
        pltpu.make_async_copy(wg_hbm.at[pl.ds(step * (TOK // 2), TOK // 2)],
                              wgv.at[sl], dsem.at[sl]).wait()
        pltpu.make_async_copy(pg_hbm.at[pl.ds(step * (TOK // 8), TOK // 8)],
                              pgv.at[sl], dsem.at[sl]).wait()

    @pl.when(b == 0)
    def _():
        fetch(0, slot)

    @pl.when(b + 1 < BH)
    def _():
        fetch(b + 1, nslot)

    drain(b, slot)
    vw = wgv[slot].astype(jnp.bfloat16)
    vp = pgv[slot].astype(jnp.bfloat16)
    curw = jnp.concatenate([vw[:, :64], vw[:, 64:]], axis=0)
    curp = jnp.concatenate([vp[:, 16 * q:16 * (q + 1)] for q in range(8)],
                           axis=0)
    cur = jnp.concatenate(
        [curw[:, :WD], curp[:, :2 * PD],
         jnp.zeros((TOK, 4), jnp.bfloat16)], axis=1)
    zrow = jnp.zeros((1, 64), jnp.bfloat16)
    prev = jnp.concatenate([zrow, cur[:-1, :]], axis=0)
    nxt = jnp.concatenate([cur[1:, :], zrow], axis=0)
    rid = lax.broadcasted_iota(jnp.int32, (TOK, 1), 0)
    zb = jnp.zeros((), jnp.bfloat16)
    prev = jnp.where(rid % L == 0, zb, prev)
    nxt = jnp.where(rid % L == (L - 1), zb, nxt)
    e = jnp.concatenate([prev, cur, nxt], axis=1)
    y = lax.dot_general(e, w_ref[...], (((1,), (0,)), ((), ())),
                        preferred_element_type=jnp.float32)
    y = (y + cb_ref[...]).astype(jnp.bfloat16)

    mask = mask_ref[...]
    y3 = y.reshape(SPB, L, H)
    pieces = []
    for j in range(3):
        bias = jnp.where(mask == j + 1, 0.0, -1e4).astype(
            jnp.bfloat16)[:, :, None]
        pieces.append(jnp.max(y3 + bias, axis=1))
    feat = jnp.tanh(
        jnp.concatenate(pieces, axis=1).astype(jnp.float32))

    r = xrel_ref[b]
    rel = relw_ref[pl.ds(r, 1), :]
    scores = lax.dot_general(feat, rel, (((1,), (1,)), ((), ())),
                             preferred_element_type=jnp.float32)
    m = jnp.max(scores, axis=0, keepdims=True)
    ex = jnp.exp(scores - m)
    att = ex / jnp.sum(ex, axis=0, keepdims=True)
    bag = lax.dot_general(att, feat, (((0,), (0,)), ((), ())),
                          preferred_element_type=jnp.float32)
    logits = lax.dot_general(bag, relwt_ref[...], (((1,), (0,)), ((), ())),
                             preferred_element_type=jnp.float32)
    out_ref[...] = (logits + relb_ref[...]).reshape(1, 1, R)


def _encode_attend(xrel, wg, pg, mask2d, wfull, cb2, relw, relwt, relb2):
    out3 = pl.pallas_call(
        _tc_body,
        grid_spec=pltpu.PrefetchScalarGridSpec(
            num_scalar_prefetch=1,
            grid=(BH,),
            in_specs=[
                pl.BlockSpec(memory_space=pltpu.MemorySpace.HBM),
                pl.BlockSpec(memory_space=pltpu.MemorySpace.HBM),
                pl.BlockSpec((SPB, L), lambda b, *_: (b, 0)),
                pl.BlockSpec((192, H), lambda b, *_: (0, 0)),
                pl.BlockSpec((1, H), lambda b, *_: (0, 0)),
                pl.BlockSpec((R, 3 * H), lambda b, *_: (0, 0)),
                pl.BlockSpec((3 * H, R), lambda b, *_: (0, 0)),
                pl.BlockSpec((1, R), lambda b, *_: (0, 0)),
            ],
            out_specs=pl.BlockSpec((1, 1, R), lambda b, *_: (b, 0, 0)),
            scratch_shapes=[
                pltpu.VMEM((2, TOK // 2, 128), jnp.float32),
                pltpu.VMEM((2, TOK // 8, 128), jnp.float32),
                pltpu.SemaphoreType.DMA((2,)),
            ],
        ),
        out_shape=jax.ShapeDtypeStruct((BH, 1, R), jnp.float32),
        compiler_params=pltpu.CompilerParams(
            dimension_semantics=("arbitrary",)),
    )(xrel, wg, pg, mask2d, wfull, cb2, relw, relwt, relb2)
    return out3.reshape(BH, R)


def kernel(X, X_Pos1, X_Pos2, X_Mask, X_Scope, X_Rel, word_emb, pos1_emb,
           pos2_emb, conv_w, conv_b, rel_w, rel_b):
    wtab = jnp.pad(word_emb, ((0, 0), (0, 64 - WD)))
    PL = pos1_emb.shape[0]
    ptab = jnp.concatenate(
        [jnp.broadcast_to(pos1_emb[:, None, :], (PL, PL, PD)),
         jnp.broadcast_to(pos2_emb[None, :, :], (PL, PL, PD)),
         jnp.zeros((PL, PL, 16 - 2 * PD), jnp.float32)],
        axis=-1).reshape(PL * PL, 16)
    xw = X.astype(jnp.int32).reshape(NL // CH, CH)
    xp = (X_Pos1.astype(jnp.int32) * PL
          + X_Pos2.astype(jnp.int32)).reshape(NL // CH, CH)
    gathered = [
        _sc_gather(wtab, ptab,
                   xw[h * (NLH // CH):(h + 1) * (NLH // CH)],
                   xp[h * (NLH // CH):(h + 1) * (NLH // CH)])
        for h in range(HALVES)
    ]

    mask2d = X_Mask.astype(jnp.int32)
    wblocks = [
        jnp.concatenate([conv_w[k], jnp.zeros((4, H), jnp.float32)], axis=0)
        for k in range(3)
    ]
    wfull = jnp.concatenate(wblocks, axis=0).astype(jnp.bfloat16)
    cb2 = conv_b.reshape(1, H)
    relwt = rel_w.T
    relb2 = rel_b.reshape(1, R)
    xrel = X_Rel.astype(jnp.int32)
    outs = [
        _encode_attend(xrel[h * BH:(h + 1) * BH], gathered[h][0],
                       gathered[h][1], mask2d[h * (N // HALVES):
                                              (h + 1) * (N // HALVES)],
                       wfull, cb2, rel_w, relwt, relb2)
        for h in range(HALVES)
    ]
    return jnp.concatenate(outs, axis=0)


# speedup vs baseline: 1.0542x; 1.0542x over previous
import functools

import jax
import jax.numpy as jnp
from jax import lax
from jax.experimental import pallas as pl
from jax.experimental.pallas import tpu as pltpu
from jax.experimental.pallas import tpu_sc as plsc

N = 1024
L = 256
B = 128
V = 100000
WD = 50
PD = 5
H = 230
R = 53
SPB = N // B
TOK = SPB * L

NL = N * L
HALVES = 4
NLH = NL // HALVES
BH = B // HALVES
NW = 32
TPW = NLH // NW
CH = 128
SR = 4
BLK = CH * SR
NB = TPW // BLK
NIR = TPW // CH

WROWS = NLH // 2
PROWS = NLH // 8


def _fire(block, wtab, ptab, idxw, idxp, bw, bp, sem):
    for r in range(SR):
        pltpu.async_copy(wtab.at[idxw.at[block * SR + r]],
                         bw.at[pl.ds(r * CH, CH)], sem)
        pltpu.async_copy(ptab.at[idxp.at[block * SR + r]],
                         bp.at[pl.ds(r * CH, CH)], sem)


def _drain(wtab, ptab, bw, bp, sem):
    for r in range(SR):
        pltpu.make_async_copy(wtab.at[pl.ds(0, CH)],
                              bw.at[pl.ds(r * CH, CH)], sem).wait()
        pltpu.make_async_copy(ptab.at[pl.ds(0, CH)],
                              bp.at[pl.ds(r * CH, CH)], sem).wait()


def _wb(wid, block, bw, bp, wg, pg):
    off = wid * TPW + block * BLK
    btc = off // TOK
    p = off % TOK
    rw = btc * (TOK // 2) + p % (TOK // 2)
    qw = p // (TOK // 2)
    pltpu.sync_copy(bw, wg.at[pl.ds(rw, BLK), pl.ds(qw * 64, 64)])
    rp = btc * (TOK // 8)
    q8 = p // (TOK // 8)
    pltpu.sync_copy(bp.at[pl.ds(0, 256)],
                    pg.at[pl.ds(rp, 256), pl.ds(q8 * 16, 16)])
    pltpu.sync_copy(bp.at[pl.ds(256, 256)],
                    pg.at[pl.ds(rp, 256), pl.ds((q8 + 1) * 16, 16)])


@functools.partial(
    pl.kernel,
    mesh=plsc.VectorSubcoreMesh(core_axis_name="c", subcore_axis_name="s"),
    compiler_params=pltpu.CompilerParams(use_tc_tiling_on_sc=False),
    out_type=[
        jax.ShapeDtypeStruct((WROWS, 128), jnp.float32),
        jax.ShapeDtypeStruct((PROWS, 128), jnp.float32),
    ],
    scratch_types=[
        pltpu.VMEM((NIR, CH), jnp.int32),
        pltpu.VMEM((NIR, CH), jnp.int32),
        pltpu.VMEM((BLK, 64), jnp.float32),
        pltpu.VMEM((BLK, 64), jnp.float32),
        pltpu.VMEM((BLK, 16), jnp.float32),
        pltpu.VMEM((BLK, 16), jnp.float32),
        pltpu.SemaphoreType.DMA,
    ],
)
def _sc_gather(wtab, ptab, xw, xp, wg, pg,
               idxw, idxp, bwa, bwb, bpa, bpb, sem):
    wid = lax.axis_index("s") * 2 + lax.axis_index("c")
    pltpu.sync_copy(xw.at[pl.ds(wid * NIR, NIR)], idxw)
    pltpu.sync_copy(xp.at[pl.ds(wid * NIR, NIR)], idxp)
    _fire(0, wtab, ptab, idxw, idxp, bwa, bpa, sem)

    def body(k, carry):
        _fire(2 * k + 1, wtab, ptab, idxw, idxp, bwb, bpb, sem)
        _drain(wtab, ptab, bwa, bpa, sem)
        _wb(wid, 2 * k, bwa, bpa, wg, pg)

        @pl.when(k < NB // 2 - 1)
        def _():
            _fire(2 * k + 2, wtab, ptab, idxw, idxp, bwa, bpa, sem)

        _drain(wtab, ptab, bwb, bpb, sem)
        _wb(wid, 2 * k + 1, bwb, bpb, wg, pg)
        return carry

    lax.fori_loop(0, NB // 2, body, 0)


def _tc_body(xrel_ref, wg_hbm, pg_hbm, mask_ref, w_ref, cb_ref,
             relw_ref, relwt_ref, relb_ref, out_ref, wgv, pgv, dsem):
    b = pl.program_id(0)
    slot = lax.rem(b, 2)
    nslot = lax.rem(b + 1, 2)

    def fetch(step, sl):
        pltpu.make_async_copy(wg_hbm.at[pl.ds(step * (TOK // 2), TOK // 2)],
                              wgv.at[sl], dsem.at[sl]).start()
        pltpu.make_async_copy(pg_hbm.at[pl.ds(step * (TOK // 8), TOK // 8)],
                              pgv.at[sl], dsem.at[sl]).start()

    def drain(step, sl):
        pltpu.make_async_copy(wg_hbm.at[pl.ds(step * (TOK // 2), TOK // 2)],
                              wgv.at[sl], dsem.at[sl]).wait()
        pltpu.make_async_copy(pg_hbm.at[pl.ds(step * (TOK // 8), TOK // 8)],
                              pgv.at[sl], dsem.at[sl]).wait()

    @pl.when(b == 0)
    def _():
        fetch(0, slot)

    @pl.when(b + 1 < BH)
    def _():
        fetch(b + 1, nslot)

    drain(b, slot)
    vw = wgv[slot].astype(jnp.bfloat16)
    vp = pgv[slot].astype(jnp.bfloat16)
    curw = jnp.concatenate([vw[:, :64], vw[:, 64:]], axis=0)
    curp = jnp.concatenate([vp[:, 16 * q:16 * (q + 1)] for q in range(8)],
                           axis=0)
    cur = jnp.concatenate(
        [curw[:, :WD], curp[:, :2 * PD],
         jnp.zeros((TOK, 4), jnp.bfloat16)], axis=1)
    zrow = jnp.zeros((1, 64), jnp.bfloat16)
    prev = jnp.concatenate([zrow, cur[:-1, :]], axis=0)
    nxt = jnp.concatenate([cur[1:, :], zrow], axis=0)
    rid = lax.broadcasted_iota(jnp.int32, (TOK, 1), 0)
    zb = jnp.zeros((), jnp.bfloat16)
    prev = jnp.where(rid % L == 0, zb, prev)
    nxt = jnp.where(rid % L == (L - 1), zb, nxt)
    e = jnp.concatenate([prev, cur, nxt], axis=1)
    y = lax.dot_general(e, w_ref[...], (((1,), (0,)), ((), ())),
                        preferred_element_type=jnp.float32)
    y = (y + cb_ref[...]).astype(jnp.bfloat16)

    mask = mask_ref[...]
    y3 = y.reshape(SPB, L, H)
    pieces = []
    for j in range(3):
        bias = jnp.where(mask == j + 1, 0.0, -1e4).astype(
            jnp.bfloat16)[:, :, None]
        pieces.append(jnp.max(y3 + bias, axis=1))
    feat = jnp.tanh(
        jnp.concatenate(pieces, axis=1).astype(jnp.float32))

    r = xrel_ref[b]
    rel = relw_ref[pl.ds(r, 1), :]
    scores = lax.dot_general(feat, rel, (((1,), (1,)), ((), ())),
                             preferred_element_type=jnp.float32)
    m = jnp.max(scores, axis=0, keepdims=True)
    ex = jnp.exp(scores - m)
    att = ex / jnp.sum(ex, axis=0, keepdims=True)
    bag = lax.dot_general(att, feat, (((0,), (0,)), ((), ())),
                          preferred_element_type=jnp.float32)
    logits = lax.dot_general(bag, relwt_ref[...], (((1,), (0,)), ((), ())),
                             preferred_element_type=jnp.float32)
    out_ref[...] = (logits + relb_ref[...]).reshape(1, 1, R)


def _encode_attend(xrel, wg, pg, mask2d, wfull, cb2, relw, relwt, relb2):
    out3 = pl.pallas_call(
        _tc_body,
        grid_spec=pltpu.PrefetchScalarGridSpec(
            num_scalar_prefetch=1,
            grid=(BH,),
            in_specs=[
                pl.BlockSpec(memory_space=pltpu.MemorySpace.HBM),
                pl.BlockSpec(memory_space=pltpu.MemorySpace.HBM),
                pl.BlockSpec((SPB, L), lambda b, *_: (b, 0)),
                pl.BlockSpec((192, H), lambda b, *_: (0, 0)),
                pl.BlockSpec((1, H), lambda b, *_: (0, 0)),
                pl.BlockSpec((R, 3 * H), lambda b, *_: (0, 0)),
                pl.BlockSpec((3 * H, R), lambda b, *_: (0, 0)),
                pl.BlockSpec((1, R), lambda b, *_: (0, 0)),
            ],
            out_specs=pl.BlockSpec((1, 1, R), lambda b, *_: (b, 0, 0)),
            scratch_shapes=[
                pltpu.VMEM((2, TOK // 2, 128), jnp.float32),
                pltpu.VMEM((2, TOK // 8, 128), jnp.float32),
                pltpu.SemaphoreType.DMA((2,)),
            ],
        ),
        out_shape=jax.ShapeDtypeStruct((BH, 1, R), jnp.float32),
        compiler_params=pltpu.CompilerParams(
            dimension_semantics=("arbitrary",)),
    )(xrel, wg, pg, mask2d, wfull, cb2, relw, relwt, relb2)
    return out3.reshape(BH, R)


def kernel(X, X_Pos1, X_Pos2, X_Mask, X_Scope, X_Rel, word_emb, pos1_emb,
           pos2_emb, conv_w, conv_b, rel_w, rel_b):
    wtab = jnp.pad(word_emb, ((0, 0), (0, 64 - WD)))
    PL = pos1_emb.shape[0]
    ptab = jnp.concatenate(
        [jnp.broadcast_to(pos1_emb[:, None, :], (PL, PL, PD)),
         jnp.broadcast_to(pos2_emb[None, :, :], (PL, PL, PD)),
         jnp.zeros((PL, PL, 16 - 2 * PD), jnp.float32)],
        axis=-1).reshape(PL * PL, 16)
    xw = X.astype(jnp.int32).reshape(NL // CH, CH)
    xp = (X_Pos1.astype(jnp.int32) * PL
          + X_Pos2.astype(jnp.int32)).reshape(NL // CH, CH)
    gathered = [
        _sc_gather(wtab, ptab,
                   xw[h * (NLH // CH):(h + 1) * (NLH // CH)],
                   xp[h * (NLH // CH):(h + 1) * (NLH // CH)])
        for h in range(HALVES)
    ]

    mask2d = X_Mask.astype(jnp.int32)
    wblocks = [
        jnp.concatenate([conv_w[k], jnp.zeros((4, H), jnp.float32)], axis=0)
        for k in range(3)
    ]
    wfull = jnp.concatenate(wblocks, axis=0).astype(jnp.bfloat16)
    cb2 = conv_b.reshape(1, H)
    relwt = rel_w.T
    relb2 = rel_b.reshape(1, R)
    xrel = X_Rel.astype(jnp.int32)
    outs = [
        _encode_attend(xrel[h * BH:(h + 1) * BH], gathered[h][0],
                       gathered[h][1], mask2d[h * (N // HALVES):
                                              (h + 1) * (N // HALVES)],
                       wfull, cb2, rel_w, relwt, relb2)
        for h in range(HALVES)
    ]
    return jnp.concatenate(outs, axis=0)
